# baseline (speedup 1.0000x reference)
import functools

import jax
import jax.numpy as jnp
from jax import lax
from jax.experimental import pallas as pl
from jax.experimental.pallas import tpu as pltpu
from jax.experimental.pallas import tpu_sc as plsc

_B_BLK = 2048
_C_BLK = 512
_P_BLK = 512
_N_COMPOSE = 3
_HIDDEN = 256
_EPS = 1e-6


def _make_route_sc(batch, n_slots):
    info = plsc.get_sparse_core_info()
    n_workers = info.num_cores * info.num_subcores
    rows_per = batch // n_workers
    n_groups = rows_per // 16
    mesh = plsc.VectorSubcoreMesh(core_axis_name="c", subcore_axis_name="s")
    neg = jnp.float32(-1e30)

    @functools.partial(
        pl.kernel, mesh=mesh,
        out_type=jax.ShapeDtypeStruct((_N_COMPOSE, n_slots, batch),
                                      jnp.float32),
        scratch_types=[
            pltpu.VMEM((n_slots, rows_per), jnp.float32),
            pltpu.VMEM((_N_COMPOSE, n_slots, rows_per), jnp.float32),
        ],
    )
    def route(swt_hbm, out_hbm, cols_v, wout_v):
        wid = lax.axis_index("s") * info.num_cores + lax.axis_index("c")
        base = wid * rows_per
        pltpu.sync_copy(swt_hbm.at[:, pl.ds(base, rows_per)], cols_v)
        iota = lax.iota(jnp.int32, 16)
        big = iota * 0 + 99

        def body(g, carry):
            off = g * 16
            vs = [cols_v[s, pl.ds(off, 16)] for s in range(n_slots)]
            ms, idxs = [], []
            for _ in range(_N_COMPOSE):
                m = vs[0]
                for s in range(1, n_slots):
                    m = jnp.maximum(m, vs[s])
                idx = big
                for s in range(n_slots - 1, -1, -1):
                    idx = jnp.where(vs[s] == m, s, idx)
                ms.append(m)
                idxs.append(idx)
                vs = [jnp.where(idx == s, neg, vs[s])
                      for s in range(n_slots)]
            es = [jnp.exp(m - ms[0]) for m in ms]
            denom = es[0] + es[1] + es[2]
            for t in range(_N_COMPOSE):
                mix_t = es[t] / denom
                for s in range(n_slots):
                    wout_v[t, s, pl.ds(off, 16)] = jnp.where(
                        idxs[t] == s, mix_t, 0.0)
            return carry

        lax.fori_loop(0, n_groups, body, 0)
        pltpu.sync_copy(wout_v, out_hbm.at[:, :, pl.ds(base, rows_per)])

    return route


def _fuse_kern(a_ref, b_ref, out_ref):
    out_ref[...] = jnp.dot(a_ref[...], b_ref[...],
                           preferred_element_type=jnp.float32
                           ).astype(jnp.bfloat16)


def _kern(wm_ref, emb_ref, wih_ref, whh_ref, bih_ref, bhh_ref,
          w1_ref, b1_ref, g1_ref, sww1_ref, sww3_ref, fw_ref,
          b2_ref, g2_ref, out_ref, x_scr):
    c = pl.program_id(1)
    n_c = pl.num_programs(1)

    @pl.when(c == 0)
    def _routing_gru_proj():
        emb = emb_ref[...]
        whh = whh_ref[...]
        bih = bih_ref[...]
        bhh = bhh_ref[...]
        emb_wih = jnp.dot(emb, wih_ref[...],
                          preferred_element_type=jnp.float32
                          ).astype(jnp.bfloat16)
        for sb in range(_B_BLK // _P_BLK):
            rs = pl.ds(sb * _P_BLK, _P_BLK)
            wm = wm_ref[:, :, rs]
            h = jnp.zeros((_P_BLK, _HIDDEN), jnp.float32)
            for t in range(_N_COMPOSE):
                w_oh_t = wm[t].astype(jnp.bfloat16)
                gi = jax.lax.dot_general(
                    w_oh_t, emb_wih, (((0,), (0,)), ((), ())),
                    preferred_element_type=jnp.float32) + bih
                gh = jnp.dot(h.astype(jnp.bfloat16), whh,
                             preferred_element_type=jnp.float32) + bhh
                i_r, i_z, i_n = jnp.split(gi, 3, axis=-1)
                h_r, h_z, h_n = jnp.split(gh, 3, axis=-1)
                r = jax.nn.sigmoid(i_r + h_r)
                z = jax.nn.sigmoid(i_z + h_z)
                n = jnp.tanh(i_n + r * h_n)
                h = (1.0 - z) * n + z * h
            x = jnp.dot(h.astype(jnp.bfloat16), w1_ref[...],
                        preferred_element_type=jnp.float32) + b1_ref[...]
            x = x * jax.lax.rsqrt(
                jnp.mean(x * x, axis=-1, keepdims=True) + _EPS) * g1_ref[...]
            x_scr[rs, :] = x.astype(jnp.bfloat16)

    xb = x_scr[...]
    a = jnp.dot(xb, sww1_ref[...], preferred_element_type=jnp.float32)
    b = jnp.dot(xb, sww3_ref[...], preferred_element_type=jnp.float32)
    mid = (jax.nn.silu(a) * b).astype(jnp.bfloat16)
    contrib = jnp.dot(mid, fw_ref[...], preferred_element_type=jnp.float32)

    @pl.when(c == 0)
    def _init_acc():
        out_ref[...] = contrib

    @pl.when(c > 0)
    def _accum():
        out_ref[...] += contrib

    @pl.when(c == n_c - 1)
    def _finish():
        y = out_ref[...] + b2_ref[...]
        y = y * jax.lax.rsqrt(
            jnp.mean(y * y, axis=-1, keepdims=True) + _EPS) * g2_ref[...]
        out_ref[...] = y


def kernel(skill_weights, skill_embeddings, W_ih, W_hh, b_ih, b_hh,
           gp_w1, gp_b1, gp_g1, sw_w1, sw_w3, sw_w2, gp_w2, gp_b2, gp_g2):
    batch, n_slots = skill_weights.shape
    embed = skill_embeddings.shape[1]
    dim = gp_w1.shape[1]
    inner = sw_w1.shape[1]
    goal = gp_w2.shape[1]
    n_b = batch // _B_BLK

    bf = jnp.bfloat16
    emb = skill_embeddings.astype(bf)
    wih_t = W_ih.T.astype(bf)
    whh_t = W_hh.T.astype(bf)
    bih = b_ih.reshape(1, -1)
    bhh = b_hh.reshape(1, -1)
    w1 = gp_w1.astype(bf)
    b1 = gp_b1.reshape(1, -1)
    g1 = gp_g1.reshape(1, -1)
    n_c = inner // _C_BLK
    sw1 = sw_w1.astype(bf)
    sw3 = sw_w3.astype(bf)
    b2 = gp_b2.reshape(1, -1)
    g2 = gp_g2.reshape(1, -1)

    fw = pl.pallas_call(
        _fuse_kern,
        out_shape=jax.ShapeDtypeStruct((inner, goal), jnp.bfloat16),
    )(sw_w2.astype(bf), gp_w2.astype(bf))

    three_h = wih_t.shape[1]

    wmix = _make_route_sc(batch, n_slots)(skill_weights.T)

    grid = (n_b, n_c)
    const = lambda i, c: (0, 0)
    out = pl.pallas_call(
        _kern,
        grid=grid,
        in_specs=[
            pl.BlockSpec((_N_COMPOSE, n_slots, _B_BLK),
                         lambda i, c: (0, 0, i)),
            pl.BlockSpec((n_slots, embed), const),
            pl.BlockSpec((embed, three_h), const),
            pl.BlockSpec((_HIDDEN, three_h), const),
            pl.BlockSpec((1, three_h), const),
            pl.BlockSpec((1, three_h), const),
            pl.BlockSpec((_HIDDEN, dim), const),
            pl.BlockSpec((1, dim), const),
            pl.BlockSpec((1, dim), const),
            pl.BlockSpec((dim, _C_BLK), lambda i, c: (0, c)),
            pl.BlockSpec((dim, _C_BLK), lambda i, c: (0, c)),
            pl.BlockSpec((_C_BLK, goal), lambda i, c: (c, 0)),
            pl.BlockSpec((1, goal), const),
            pl.BlockSpec((1, goal), const),
        ],
        out_specs=pl.BlockSpec((_B_BLK, goal), lambda i, c: (i, 0)),
        out_shape=jax.ShapeDtypeStruct((batch, goal), jnp.float32),
        scratch_shapes=[
            pltpu.VMEM((_B_BLK, dim), jnp.bfloat16),
        ],
        compiler_params=pltpu.CompilerParams(
            dimension_semantics=("arbitrary", "arbitrary"),
        ),
    )(wmix, emb, wih_t, whh_t, bih, bhh,
      w1, b1, g1, sw1, sw3, fw, b2, g2)
    return out


# speedup vs baseline: 1.0929x; 1.0929x over previous
import functools

import jax
import jax.numpy as jnp
from jax import lax
from jax.experimental import pallas as pl
from jax.experimental.pallas import tpu as pltpu
from jax.experimental.pallas import tpu_sc as plsc

_B_BLK = 1024
_C_BLK = 1024
_N_COMPOSE = 3
_HIDDEN = 256
_EPS = 1e-6


def _make_route_sc(batch, n_slots):
    info = plsc.get_sparse_core_info()
    n_workers = info.num_cores * info.num_subcores
    rows_per = batch // n_workers
    n_groups = rows_per // 16
    mesh = plsc.VectorSubcoreMesh(core_axis_name="c", subcore_axis_name="s")
    neg = jnp.float32(-1e30)

    @functools.partial(
        pl.kernel, mesh=mesh,
        out_type=jax.ShapeDtypeStruct((_N_COMPOSE, n_slots, batch),
                                      jnp.float32),
        scratch_types=[
            pltpu.VMEM((n_slots, rows_per), jnp.float32),
            pltpu.VMEM((_N_COMPOSE, n_slots, rows_per), jnp.float32),
        ],
    )
    def route(swt_hbm, out_hbm, cols_v, wout_v):
        wid = lax.axis_index("s") * info.num_cores + lax.axis_index("c")
        base = wid * rows_per
        pltpu.sync_copy(swt_hbm.at[:, pl.ds(base, rows_per)], cols_v)
        iota = lax.iota(jnp.int32, 16)
        big = iota * 0 + 99

        def body(g, carry):
            off = g * 16
            vs = [cols_v[s, pl.ds(off, 16)] for s in range(n_slots)]
            ms, idxs = [], []
            for _ in range(_N_COMPOSE):
                m = vs[0]
                for s in range(1, n_slots):
                    m = jnp.maximum(m, vs[s])
                idx = big
                for s in range(n_slots - 1, -1, -1):
                    idx = jnp.where(vs[s] == m, s, idx)
                ms.append(m)
                idxs.append(idx)
                vs = [jnp.where(idx == s, neg, vs[s])
                      for s in range(n_slots)]
            es = [jnp.exp(m - ms[0]) for m in ms]
            denom = es[0] + es[1] + es[2]
            for t in range(_N_COMPOSE):
                mix_t = es[t] / denom
                for s in range(n_slots):
                    wout_v[t, s, pl.ds(off, 16)] = jnp.where(
                        idxs[t] == s, mix_t, 0.0)
            return carry

        lax.fori_loop(0, n_groups, body, 0)
        pltpu.sync_copy(wout_v, out_hbm.at[:, :, pl.ds(base, rows_per)])

    return route


def _fuse_kern(a_ref, b_ref, out_ref):
    out_ref[...] = jnp.dot(a_ref[...], b_ref[...],
                           preferred_element_type=jnp.float32
                           ).astype(jnp.bfloat16)


def _kern(wm_ref, emb_ref, wih_ref, whh_ref, bih_ref, bhh_ref,
          w1_ref, b1_ref, g1_ref, sww1_ref, sww3_ref, fw_ref,
          b2_ref, g2_ref, out_ref, x_scr, acc_scr):
    c = pl.program_id(1)
    n_c = pl.num_programs(1)

    @pl.when(c == 0)
    def _routing_gru_proj():
        wm = wm_ref[...]
        emb = emb_ref[...]
        h = jnp.zeros((wm.shape[2], _HIDDEN), jnp.float32)
        whh = whh_ref[...]
        bih = bih_ref[...]
        bhh = bhh_ref[...]
        emb_wih = jnp.dot(emb, wih_ref[...],
                          preferred_element_type=jnp.float32
                          ).astype(jnp.bfloat16)
        for t in range(_N_COMPOSE):
            w_oh_t = wm[t].astype(jnp.bfloat16)
            gi = jax.lax.dot_general(
                w_oh_t, emb_wih, (((0,), (0,)), ((), ())),
                preferred_element_type=jnp.float32) + bih
            gh = jnp.dot(h.astype(jnp.bfloat16), whh,
                         preferred_element_type=jnp.float32) + bhh
            i_r, i_z, i_n = jnp.split(gi, 3, axis=-1)
            h_r, h_z, h_n = jnp.split(gh, 3, axis=-1)
            r = jax.nn.sigmoid(i_r + h_r)
            z = jax.nn.sigmoid(i_z + h_z)
            n = jnp.tanh(i_n + r * h_n)
            h = (1.0 - z) * n + z * h
        x = jnp.dot(h.astype(jnp.bfloat16), w1_ref[...],
                    preferred_element_type=jnp.float32) + b1_ref[...]
        x = x * jax.lax.rsqrt(
            jnp.mean(x * x, axis=-1, keepdims=True) + _EPS) * g1_ref[...]
        x_scr[...] = x.astype(jnp.bfloat16)

    xb = x_scr[...]
    a = jnp.dot(xb, sww1_ref[...], preferred_element_type=jnp.float32)
    b = jnp.dot(xb, sww3_ref[...], preferred_element_type=jnp.float32)
    mid = (jax.nn.silu(a) * b).astype(jnp.bfloat16)
    contrib = jnp.dot(mid, fw_ref[...], preferred_element_type=jnp.float32)

    @pl.when(c == 0)
    def _init_acc():
        acc_scr[...] = contrib

    @pl.when(c > 0)
    def _accum():
        acc_scr[...] += contrib

    @pl.when(c == n_c - 1)
    def _finish():
        y = acc_scr[...] + b2_ref[...]
        y = y * jax.lax.rsqrt(
            jnp.mean(y * y, axis=-1, keepdims=True) + _EPS) * g2_ref[...]
        out_ref[...] = y


def kernel(skill_weights, skill_embeddings, W_ih, W_hh, b_ih, b_hh,
           gp_w1, gp_b1, gp_g1, sw_w1, sw_w3, sw_w2, gp_w2, gp_b2, gp_g2):
    batch, n_slots = skill_weights.shape
    embed = skill_embeddings.shape[1]
    dim = gp_w1.shape[1]
    inner = sw_w1.shape[1]
    goal = gp_w2.shape[1]
    n_b = batch // _B_BLK

    bf = jnp.bfloat16
    emb = skill_embeddings.astype(bf)
    wih_t = W_ih.T.astype(bf)
    whh_t = W_hh.T.astype(bf)
    bih = b_ih.reshape(1, -1)
    bhh = b_hh.reshape(1, -1)
    w1 = gp_w1.astype(bf)
    b1 = gp_b1.reshape(1, -1)
    g1 = gp_g1.reshape(1, -1)
    n_c = inner // _C_BLK
    sw1 = sw_w1.astype(bf)
    sw3 = sw_w3.astype(bf)
    b2 = gp_b2.reshape(1, -1)
    g2 = gp_g2.reshape(1, -1)

    fw = pl.pallas_call(
        _fuse_kern,
        out_shape=jax.ShapeDtypeStruct((inner, goal), jnp.bfloat16),
    )(sw_w2.astype(bf), gp_w2.astype(bf))

    three_h = wih_t.shape[1]

    wmix = _make_route_sc(batch, n_slots)(skill_weights.T)

    grid = (n_b, n_c)
    const = lambda i, c: (0, 0)
    out = pl.pallas_call(
        _kern,
        grid=grid,
        in_specs=[
            pl.BlockSpec((_N_COMPOSE, n_slots, _B_BLK),
                         lambda i, c: (0, 0, i)),
            pl.BlockSpec((n_slots, embed), const),
            pl.BlockSpec((embed, three_h), const),
            pl.BlockSpec((_HIDDEN, three_h), const),
            pl.BlockSpec((1, three_h), const),
            pl.BlockSpec((1, three_h), const),
            pl.BlockSpec((_HIDDEN, dim), const),
            pl.BlockSpec((1, dim), const),
            pl.BlockSpec((1, dim), const),
            pl.BlockSpec((dim, _C_BLK), lambda i, c: (0, c)),
            pl.BlockSpec((dim, _C_BLK), lambda i, c: (0, c)),
            pl.BlockSpec((_C_BLK, goal), lambda i, c: (c, 0)),
            pl.BlockSpec((1, goal), const),
            pl.BlockSpec((1, goal), const),
        ],
        out_specs=pl.BlockSpec((_B_BLK, goal), lambda i, c: (i, 0)),
        out_shape=jax.ShapeDtypeStruct((batch, goal), jnp.float32),
        scratch_shapes=[
            pltpu.VMEM((_B_BLK, dim), jnp.bfloat16),
            pltpu.VMEM((_B_BLK, goal), jnp.float32),
        ],
        compiler_params=pltpu.CompilerParams(
            dimension_semantics=("arbitrary", "arbitrary"),
        ),
    )(wmix, emb, wih_t, whh_t, bih, bhh,
      w1, b1, g1, sw1, sw3, fw, b2, g2)
    return out
